# baseline (speedup 1.0000x reference)
import functools

import jax
import jax.numpy as jnp
from jax import lax
from jax.experimental import pallas as pl
from jax.experimental.pallas import tpu as pltpu
from jax.experimental.pallas import tpu_sc as plsc

VOCAB = 1000
VMAIN = 896
VTAIL = 128
VPAD = 1024
VREM = VOCAB - VMAIN
BATCH = 1024
SEQ = 50
SEQA = 48
SEQP = 56
NUM_CORES = 2
NUM_SUBCORES = 16
NUM_WORKERS = NUM_CORES * NUM_SUBCORES
B_PER_W = BATCH // NUM_WORKERS
LANES = 16
KREM = VREM // LANES
BGRP = 4

_mesh = plsc.VectorSubcoreMesh(core_axis_name="c", subcore_axis_name="s")


@functools.partial(
    pl.kernel,
    out_type=jax.ShapeDtypeStruct((BATCH, SEQ, VOCAB), jnp.float32),
    mesh=_mesh,
    compiler_params=pltpu.CompilerParams(
        use_tc_tiling_on_sc=True, needs_layout_passes=False
    ),
    scratch_types=[
        pltpu.VMEM((B_PER_W * SEQP,), jnp.int32),
        pltpu.VMEM((B_PER_W * 2,), jnp.int32),
        pltpu.VMEM((2, SEQ, VOCAB), jnp.float32),
        pltpu.VMEM((SEQA, VTAIL), jnp.float32),
        pltpu.VMEM((2 * BGRP, VPAD), jnp.float32),
        pltpu.SemaphoreType.DMA,
        pltpu.SemaphoreType.DMA,
        pltpu.SemaphoreType.DMA,
        pltpu.SemaphoreType.DMA,
    ],
)
def _embedding_gather(
    main_hbm, tail_hbm, full_hbm, idx_hbm, idxb_hbm, out_hbm,
    idx_v, idxb_v, stag, stag_t, stag_b,
    sm0, sm1, st, sb,
):
    wid = lax.axis_index("s") * NUM_CORES + lax.axis_index("c")
    base = wid * B_PER_W
    sems_m = (sm0, sm1)

    pltpu.sync_copy(idx_hbm.at[pl.ds(base * SEQP, B_PER_W * SEQP)], idx_v)
    pltpu.sync_copy(idxb_hbm.at[pl.ds(base * 2, B_PER_W * 2)], idxb_v)

    def main_desc(c, b):
        idx48 = idx_v.at[pl.ds(c * SEQP, SEQA)]
        dst = stag.at[b].at[pl.ds(0, SEQA), pl.ds(0, VMAIN)]
        return pltpu.make_async_copy(main_hbm.at[idx48], dst, sems_m[b])

    def tail_desc(c):
        idx48 = idx_v.at[pl.ds(c * SEQP, SEQA)]
        return pltpu.make_async_copy(tail_hbm.at[idx48], stag_t, st)

    def bgather(grp):
        idx8 = idxb_v.at[pl.ds(grp * 2 * BGRP, 2 * BGRP)]
        pltpu.async_copy(full_hbm.at[idx8], stag_b, sb)
        pltpu.make_async_copy(full_hbm.at[idx8], stag_b, sb).wait()

    lane = lax.iota(jnp.int32, LANES)
    rem_cols = VMAIN + KREM * LANES + lane
    rem_mask = rem_cols < VOCAB

    def repack(c, b):
        @pl.loop(0, SEQA)
        def _(r):
            for k in range(KREM):
                stag.at[b][r, pl.ds(VMAIN + k * LANES, LANES)] = (
                    stag_t[r, pl.ds(k * LANES, LANES)]
                )
            x = stag_t[r, pl.ds(KREM * LANES, LANES)]
            row_ids = jnp.full((LANES,), r, jnp.int32)
            plsc.store_scatter(stag.at[b], [row_ids, rem_cols], x, mask=rem_mask)

        boff = (c % BGRP) * 2
        for r in range(SEQ - SEQA):
            for k in range(VMAIN // LANES + KREM):
                stag.at[b][SEQA + r, pl.ds(k * LANES, LANES)] = (
                    stag_b[boff + r, pl.ds(k * LANES, LANES)]
                )
            x = stag_b[boff + r, pl.ds((VMAIN // LANES + KREM) * LANES, LANES)]
            row_ids = jnp.full((LANES,), SEQA + r, jnp.int32)
            plsc.store_scatter(stag.at[b], [row_ids, rem_cols], x, mask=rem_mask)

    def write_out(c, b):
        pltpu.sync_copy(stag.at[b], out_hbm.at[base + c])

    bgather(0)
    main_desc(0, 0).start()
    main_desc(1, 1).start()
    tail_desc(0).start()

    @pl.loop(0, B_PER_W - 2, step=2)
    def _(g):
        for b in range(2):
            c = g + b
            if b == 0:
                @pl.when(jnp.logical_and(g % BGRP == 0, g > 0))
                def _():
                    bgather(g // BGRP)

            main_desc(c, b).wait()
            tail_desc(c).wait()
            repack(c, b)
            tail_desc(c + 1).start()
            write_out(c, b)
            main_desc(c + 2, b).start()

    g = B_PER_W - 2
    main_desc(g, 0).wait()
    tail_desc(g).wait()
    repack(g, 0)
    tail_desc(g + 1).start()
    write_out(g, 0)
    main_desc(g + 1, 1).wait()
    tail_desc(g + 1).wait()
    repack(g + 1, 1)
    write_out(g + 1, 1)


def kernel(index, token_embedding_table):
    table_main = token_embedding_table[:, :VMAIN]
    table_tail = jnp.pad(
        token_embedding_table[:, VMAIN:], ((0, 0), (0, VTAIL - VREM))
    )
    table_full = jnp.pad(
        token_embedding_table, ((0, 0), (0, VPAD - VOCAB))
    )
    idxp = jnp.pad(index, ((0, 0), (0, SEQP - SEQ))).reshape(-1)
    idxb = index[:, SEQA:SEQ].reshape(-1)
    out = _embedding_gather(table_main, table_tail, table_full, idxp, idxb)
    return out, None
